# baseline (speedup 1.0000x reference)
import functools
import math

import jax
import jax.numpy as jnp
from jax import lax
from jax.experimental import pallas as pl
from jax.experimental.pallas import tpu as pltpu
from jax.experimental.pallas import tpu_sc as plsc

VOCAB = 100000
D_MODEL = 128
BATCH = 4096
SEQ = 50
SEQ_PAD = 56

NUM_CORES = 2
NUM_SUBCORES = 16
NW = NUM_CORES * NUM_SUBCORES
B_PER_W = BATCH // NW
G = 4
N_GROUPS = B_PER_W // G
SCALE = math.sqrt(D_MODEL)


def _sc_kernel(x_hbm, table_hbm, out_hbm, idx_v, buf0, buf1, gs0, gs1, ss0, ss1):
    wid = lax.axis_index("s") * NUM_CORES + lax.axis_index("c")
    b_base = wid * B_PER_W
    pltpu.sync_copy(x_hbm.at[pl.ds(b_base, B_PER_W)], idx_v)

    def fire_gathers(g, buf, gsem):
        copies = []
        for k in range(G):
            r = g * G + k
            copies.append(
                pltpu.async_copy(table_hbm.at[idx_v.at[r]], buf.at[k, pl.ds(0, SEQ)], gsem))
        return copies

    def scale_buf(buf):
        def row_body(r, c2):
            for k in range(G):
                for c8 in range(D_MODEL // 16):
                    sl = pl.ds(c8 * 16, 16)
                    buf[k, r, sl] = buf[k, r, sl] * SCALE
            return c2
        lax.fori_loop(0, SEQ, row_body, 0, unroll=False)

    def fire_scatters(g, buf, ssem):
        pltpu.async_copy(buf, out_hbm.at[pl.ds(b_base + g * G, G)], ssem)

    def drain_scatters(buf, ssem):
        pltpu.make_async_copy(buf, out_hbm.at[pl.ds(b_base, G)], ssem).wait()

    def body(i, carry):
        g0 = i * 2
        g1 = i * 2 + 1

        @pl.when(i > 0)
        def _():
            drain_scatters(buf0, ss0)
        gcopies0 = fire_gathers(g0, buf0, gs0)

        @pl.when(i > 0)
        def _():
            drain_scatters(buf1, ss1)
        gcopies1 = fire_gathers(g1, buf1, gs1)

        for c in gcopies0:
            c.wait()
        scale_buf(buf0)
        fire_scatters(g0, buf0, ss0)

        for c in gcopies1:
            c.wait()
        scale_buf(buf1)
        fire_scatters(g1, buf1, ss1)
        return carry

    lax.fori_loop(0, N_GROUPS // 2, body, 0, unroll=False)
    drain_scatters(buf0, ss0)
    drain_scatters(buf1, ss1)


_N_CHUNKS_TC = 8
_CB = BATCH // _N_CHUNKS_TC


def _tc_relayout(in_ref, out_ref, sem):
    copies = [
        pltpu.make_async_copy(
            in_ref.at[pl.ds(c * _CB, _CB), pl.ds(0, SEQ)],
            out_ref.at[pl.ds(c * _CB, _CB)],
            sem,
        )
        for c in range(_N_CHUNKS_TC)
    ]
    for cp in copies:
        cp.start()
    for cp in copies:
        cp.wait()


@functools.partial(jax.jit)
def kernel(x, table):
    mesh = plsc.VectorSubcoreMesh(core_axis_name="c", subcore_axis_name="s")
    out56 = pl.kernel(
        _sc_kernel,
        mesh=mesh,
        out_type=jax.ShapeDtypeStruct((BATCH, SEQ_PAD, D_MODEL), jnp.float32),
        scratch_types=[
            pltpu.VMEM((B_PER_W, SEQ), jnp.int32),
            pltpu.VMEM((G, SEQ_PAD, D_MODEL), jnp.float32),
            pltpu.VMEM((G, SEQ_PAD, D_MODEL), jnp.float32),
            pltpu.SemaphoreType.DMA,
            pltpu.SemaphoreType.DMA,
            pltpu.SemaphoreType.DMA,
            pltpu.SemaphoreType.DMA,
        ],
    )(x.astype(jnp.int32), table)
    return pl.pallas_call(
        _tc_relayout,
        out_shape=jax.ShapeDtypeStruct((BATCH, SEQ, D_MODEL), jnp.float32),
        in_specs=[pl.BlockSpec(memory_space=pl.ANY)],
        out_specs=pl.BlockSpec(memory_space=pl.ANY),
        scratch_shapes=[pltpu.SemaphoreType.DMA],
    )(out56)


# speedup vs baseline: 19.6961x; 19.6961x over previous
import functools
import math

import jax
import jax.numpy as jnp
from jax import lax
from jax.experimental import pallas as pl
from jax.experimental.pallas import tpu as pltpu
from jax.experimental.pallas import tpu_sc as plsc

VOCAB = 100000
D_MODEL = 128
BATCH = 4096
SEQ = 50

NUM_CORES = 2
NUM_SUBCORES = 16
NW = NUM_CORES * NUM_SUBCORES
B_PER_W = BATCH // NW
G = 4
N_GROUPS = B_PER_W // G
SCALE = math.sqrt(D_MODEL)


def _sc_kernel(x_hbm, table_hbm, out_hbm, idx_v, buf0, buf1, gs0, gs1, ss0, ss1):
    wid = lax.axis_index("s") * NUM_CORES + lax.axis_index("c")
    b_base = wid * B_PER_W
    pltpu.sync_copy(x_hbm.at[pl.ds(b_base, B_PER_W)], idx_v)

    def fire_gathers(g, buf, gsem):
        copies = []
        for k in range(G):
            r = g * G + k
            copies.append(
                pltpu.async_copy(table_hbm.at[idx_v.at[r]], buf.at[k], gsem))
        return copies

    def scale_buf(buf):
        def row_body(r, c2):
            for k in range(G):
                for c8 in range(D_MODEL // 16):
                    sl = pl.ds(c8 * 16, 16)
                    buf[k, r, sl] = buf[k, r, sl] * SCALE
            return c2
        lax.fori_loop(0, SEQ, row_body, 0, unroll=False)

    def drain_scatter(buf, ssem):
        pltpu.make_async_copy(buf, out_hbm.at[pl.ds(b_base, G)], ssem).wait()

    def body(i, carry):
        g0 = i * 2
        g1 = i * 2 + 1

        @pl.when(i > 0)
        def _():
            drain_scatter(buf0, ss0)
        gcopies0 = fire_gathers(g0, buf0, gs0)

        @pl.when(i > 0)
        def _():
            drain_scatter(buf1, ss1)
        gcopies1 = fire_gathers(g1, buf1, gs1)

        for c in gcopies0:
            c.wait()
        scale_buf(buf0)
        pltpu.async_copy(buf0, out_hbm.at[pl.ds(b_base + g0 * G, G)], ss0)

        for c in gcopies1:
            c.wait()
        scale_buf(buf1)
        pltpu.async_copy(buf1, out_hbm.at[pl.ds(b_base + g1 * G, G)], ss1)
        return carry

    lax.fori_loop(0, N_GROUPS // 2, body, 0, unroll=False)
    drain_scatter(buf0, ss0)
    drain_scatter(buf1, ss1)


@functools.partial(jax.jit)
def kernel(x, table):
    mesh = plsc.VectorSubcoreMesh(core_axis_name="c", subcore_axis_name="s")
    return pl.kernel(
        _sc_kernel,
        mesh=mesh,
        out_type=jax.ShapeDtypeStruct((BATCH, SEQ, D_MODEL), jnp.float32),
        scratch_types=[
            pltpu.VMEM((B_PER_W, SEQ), jnp.int32),
            pltpu.VMEM((G, SEQ, D_MODEL), jnp.float32),
            pltpu.VMEM((G, SEQ, D_MODEL), jnp.float32),
            pltpu.SemaphoreType.DMA,
            pltpu.SemaphoreType.DMA,
            pltpu.SemaphoreType.DMA,
            pltpu.SemaphoreType.DMA,
        ],
    )(x.astype(jnp.int32), table)
